# baseline (speedup 1.0000x reference)
import functools

import jax
import jax.numpy as jnp
from jax import lax
from jax.experimental import pallas as pl
from jax.experimental.pallas import tpu as pltpu
from jax.experimental.pallas import tpu_sc as plsc

N_NODES = 10000
N_EDGES = 320000
D = 128
NUM_RELS = 16

NUM_CORES = 2
NUM_SUBCORES = 16
NW = NUM_CORES * NUM_SUBCORES
B_CH = 128
CH_PER_W = 80
CH_PH = 40
NUM_CHUNKS = NW * CH_PER_W
E_PAD = NUM_CHUNKS * B_CH
N_PAD = 10240
ROWS_PER_SUB = N_PAD // NUM_SUBCORES


def _xw_body(feat_ref, w_ref, xw_ref):
    xw_ref[0] = jnp.dot(feat_ref[...], w_ref[0],
                        preferred_element_type=jnp.float32)


def _make_xw(feat, weight):
    bn = 2000
    nb = N_NODES // bn
    return pl.pallas_call(
        _xw_body,
        grid=(nb, NUM_RELS),
        in_specs=[
            pl.BlockSpec((bn, D), lambda i, r: (i, 0)),
            pl.BlockSpec((1, D, D), lambda i, r: (r, 0, 0)),
        ],
        out_specs=pl.BlockSpec((1, bn, D), lambda i, r: (r, i, 0)),
        out_shape=jax.ShapeDtypeStruct((NUM_RELS, N_NODES, D), jnp.float32),
    )(feat, weight)


def _sc_body(xw_hbm, src_hbm, et_hbm, dst_hbm, zeros_hbm, part_hbm,
             et_blk, dst_blk, idx_blk, rows_a, rows_b,
             acc_shared, sem_a, sem_b):
    core = lax.axis_index("c")
    sub = lax.axis_index("s")
    wid = core * NUM_SUBCORES + sub

    r0 = sub * ROWS_PER_SUB
    pltpu.sync_copy(zeros_hbm.at[pl.ds(r0, ROWS_PER_SUB)],
                    acc_shared.at[pl.ds(r0, ROWS_PER_SUB)])
    plsc.subcore_barrier()

    def fire(j, rows, sem):
        pltpu.async_copy(xw_hbm.at[idx_blk.at[j]], rows, sem)

    def drain(rows, sem):
        pltpu.make_async_copy(xw_hbm.at[idx_blk.at[0]], rows, sem).wait()

    def scat(j, rows):
        pltpu.sync_copy(rows, acc_shared.at[dst_blk.at[j]], add=True)

    for phase in range(CH_PER_W // CH_PH):
        crow = pl.multiple_of(wid * CH_PER_W + phase * CH_PH, 8)
        pltpu.sync_copy(src_hbm.at[pl.ds(crow, CH_PH)], idx_blk)
        pltpu.sync_copy(et_hbm.at[pl.ds(crow, CH_PH)], et_blk)
        pltpu.sync_copy(dst_hbm.at[pl.ds(crow, CH_PH)], dst_blk)

        @pl.loop(0, CH_PH)
        def _(j):
            for k in range(B_CH // 16):
                sl = pl.ds(k * 16, 16)
                idx_blk[j, sl] = et_blk[j, sl] * N_NODES + idx_blk[j, sl]

        fire(0, rows_a, sem_a)
        fire(1, rows_b, sem_b)

        @pl.loop(0, CH_PH - 2, step=2)
        def _(j):
            drain(rows_a, sem_a)
            scat(j, rows_a)
            fire(j + 2, rows_a, sem_a)
            drain(rows_b, sem_b)
            scat(j + 1, rows_b)
            fire(j + 3, rows_b, sem_b)

        drain(rows_a, sem_a)
        scat(CH_PH - 2, rows_a)
        drain(rows_b, sem_b)
        scat(CH_PH - 1, rows_b)

    plsc.subcore_barrier()
    pltpu.sync_copy(acc_shared.at[pl.ds(r0, ROWS_PER_SUB)],
                    part_hbm.at[core, pl.ds(r0, ROWS_PER_SUB)])


def _run_sc(xw_flat, src, et, dst, zeros):
    mesh = plsc.VectorSubcoreMesh(core_axis_name="c", subcore_axis_name="s")
    k = pl.kernel(
        _sc_body,
        out_type=jax.ShapeDtypeStruct((NUM_CORES, N_PAD, D), jnp.float32),
        mesh=mesh,
        scratch_types=[
            pltpu.VMEM((CH_PH, B_CH), jnp.int32),
            pltpu.VMEM((CH_PH, B_CH), jnp.int32),
            pltpu.VMEM((CH_PH, B_CH), jnp.int32),
            pltpu.VMEM((B_CH, D), jnp.float32),
            pltpu.VMEM((B_CH, D), jnp.float32),
            pltpu.VMEM_SHARED((N_PAD, D), jnp.float32),
            pltpu.SemaphoreType.DMA,
            pltpu.SemaphoreType.DMA,
        ],
    )
    return k(xw_flat, src, et, dst, zeros)


def _comb_body(p_ref, feat_ref, lw_ref, b_ref, out_ref):
    out_ref[...] = (p_ref[0] + p_ref[1] + b_ref[...]
                    + jnp.dot(feat_ref[...], lw_ref[...],
                              preferred_element_type=jnp.float32))


def _combine(part, feat, loop_weight, bias2d):
    bn = 2000
    nb = N_NODES // bn
    return pl.pallas_call(
        _comb_body,
        grid=(nb,),
        in_specs=[
            pl.BlockSpec((NUM_CORES, bn, D), lambda i: (0, i, 0)),
            pl.BlockSpec((bn, D), lambda i: (i, 0)),
            pl.BlockSpec((D, D), lambda i: (0, 0)),
            pl.BlockSpec((1, D), lambda i: (0, 0)),
        ],
        out_specs=pl.BlockSpec((bn, D), lambda i: (i, 0)),
        out_shape=jax.ShapeDtypeStruct((N_NODES, D), jnp.float32),
    )(part, feat, loop_weight, bias2d)


def kernel(feat, edge_index, etypes, weight, loop_weight, bias):
    npad = E_PAD - N_EDGES
    src = jnp.concatenate(
        [edge_index[0].astype(jnp.int32), jnp.zeros((npad,), jnp.int32)]
    ).reshape(NUM_CHUNKS, B_CH)
    dst = jnp.concatenate(
        [edge_index[1].astype(jnp.int32),
         jnp.full((npad,), N_PAD - 1, jnp.int32)]
    ).reshape(NUM_CHUNKS, B_CH)
    et = jnp.concatenate(
        [etypes.astype(jnp.int32), jnp.zeros((npad,), jnp.int32)]
    ).reshape(NUM_CHUNKS, B_CH)
    zeros = jnp.zeros((N_PAD, D), jnp.float32)

    xw = _make_xw(feat, weight)
    xw_flat = xw.reshape(NUM_RELS * N_NODES, D)
    part = _run_sc(xw_flat, src, et, dst, zeros)
    bias2d = bias.reshape(1, D)
    return _combine(part, feat, loop_weight, bias2d)


# speedup vs baseline: 1.0126x; 1.0126x over previous
import functools

import jax
import jax.numpy as jnp
from jax import lax
from jax.experimental import pallas as pl
from jax.experimental.pallas import tpu as pltpu
from jax.experimental.pallas import tpu_sc as plsc

N_NODES = 10000
N_EDGES = 320000
D = 128
NUM_RELS = 16

NUM_CORES = 2
NUM_SUBCORES = 16
NW = NUM_CORES * NUM_SUBCORES
B_CH = 128
CH_PER_W = 80
CH_PH = 40
NUM_CHUNKS = NW * CH_PER_W
E_PAD = NUM_CHUNKS * B_CH
N_PAD = 10240
ROWS_PER_SUB = N_PAD // NUM_SUBCORES


def _xw_body(feat_ref, w_ref, xw_ref):
    xw_ref[0] = jnp.dot(feat_ref[...], w_ref[0],
                        preferred_element_type=jnp.float32)


def _make_xw(feat, weight):
    bn = 2000
    nb = N_NODES // bn
    return pl.pallas_call(
        _xw_body,
        grid=(nb, NUM_RELS),
        in_specs=[
            pl.BlockSpec((bn, D), lambda i, r: (i, 0)),
            pl.BlockSpec((1, D, D), lambda i, r: (r, 0, 0)),
        ],
        out_specs=pl.BlockSpec((1, bn, D), lambda i, r: (r, i, 0)),
        out_shape=jax.ShapeDtypeStruct((NUM_RELS, N_NODES, D), jnp.float32),
    )(feat, weight)


def _sc_body(xw_hbm, src_hbm, et_hbm, dst_hbm, zeros_hbm, part_hbm,
             et_a, et_b, dst_a, dst_b, idx_a, idx_b, rows_a, rows_b,
             acc_shared, sem_a, sem_b, sem_ia, sem_ib):
    core = lax.axis_index("c")
    sub = lax.axis_index("s")
    wid = core * NUM_SUBCORES + sub

    r0 = sub * ROWS_PER_SUB
    pltpu.sync_copy(zeros_hbm.at[pl.ds(r0, ROWS_PER_SUB)],
                    acc_shared.at[pl.ds(r0, ROWS_PER_SUB)])
    plsc.subcore_barrier()

    bufs = ((et_a, dst_a, idx_a, rows_a, sem_a, sem_ia),
            (et_b, dst_b, idx_b, rows_b, sem_b, sem_ib))

    def stage(j, b):
        et, dst, idx, rows, sem, sem_i = bufs[b]
        base = pl.multiple_of(j * B_CH, B_CH)
        cp_s = pltpu.async_copy(src_hbm.at[pl.ds(base, B_CH)], idx, sem_i)
        cp_e = pltpu.async_copy(et_hbm.at[pl.ds(base, B_CH)], et, sem_i)
        cp_d = pltpu.async_copy(dst_hbm.at[pl.ds(base, B_CH)], dst, sem_i)
        cp_s.wait()
        cp_e.wait()
        cp_d.wait()
        for k in range(B_CH // 16):
            sl = pl.ds(k * 16, 16)
            idx[sl] = et[sl] * N_NODES + idx[sl]
        pltpu.async_copy(xw_hbm.at[idx], rows, sem)

    def drain_scat(b):
        et, dst, idx, rows, sem, sem_i = bufs[b]
        pltpu.make_async_copy(xw_hbm.at[idx], rows, sem).wait()
        pltpu.sync_copy(rows, acc_shared.at[dst], add=True)

    stage(wid, 0)

    @pl.loop(0, CH_PER_W - 2, step=2)
    def _(j):
        stage((j + 1) * NW + wid, 1)
        drain_scat(0)
        stage((j + 2) * NW + wid, 0)
        drain_scat(1)

    stage((CH_PER_W - 1) * NW + wid, 1)
    drain_scat(0)
    drain_scat(1)

    plsc.subcore_barrier()
    pltpu.sync_copy(acc_shared.at[pl.ds(r0, ROWS_PER_SUB)],
                    part_hbm.at[core, pl.ds(r0, ROWS_PER_SUB)])


def _run_sc(xw_flat, src, et, dst, zeros):
    mesh = plsc.VectorSubcoreMesh(core_axis_name="c", subcore_axis_name="s")
    k = pl.kernel(
        _sc_body,
        out_type=jax.ShapeDtypeStruct((NUM_CORES, N_PAD, D), jnp.float32),
        mesh=mesh,
        scratch_types=[
            pltpu.VMEM((B_CH,), jnp.int32),
            pltpu.VMEM((B_CH,), jnp.int32),
            pltpu.VMEM((B_CH,), jnp.int32),
            pltpu.VMEM((B_CH,), jnp.int32),
            pltpu.VMEM((B_CH,), jnp.int32),
            pltpu.VMEM((B_CH,), jnp.int32),
            pltpu.VMEM((B_CH, D), jnp.float32),
            pltpu.VMEM((B_CH, D), jnp.float32),
            pltpu.VMEM_SHARED((N_PAD, D), jnp.float32),
            pltpu.SemaphoreType.DMA,
            pltpu.SemaphoreType.DMA,
            pltpu.SemaphoreType.DMA,
            pltpu.SemaphoreType.DMA,
        ],
    )
    return k(xw_flat, src, et, dst, zeros)


def _comb_body(p_ref, feat_ref, lw_ref, b_ref, out_ref):
    out_ref[...] = (p_ref[0] + p_ref[1] + b_ref[...]
                    + jnp.dot(feat_ref[...], lw_ref[...],
                              preferred_element_type=jnp.float32))


def _combine(part, feat, loop_weight, bias2d):
    bn = 2000
    nb = N_NODES // bn
    return pl.pallas_call(
        _comb_body,
        grid=(nb,),
        in_specs=[
            pl.BlockSpec((NUM_CORES, bn, D), lambda i: (0, i, 0)),
            pl.BlockSpec((bn, D), lambda i: (i, 0)),
            pl.BlockSpec((D, D), lambda i: (0, 0)),
            pl.BlockSpec((1, D), lambda i: (0, 0)),
        ],
        out_specs=pl.BlockSpec((bn, D), lambda i: (i, 0)),
        out_shape=jax.ShapeDtypeStruct((N_NODES, D), jnp.float32),
    )(part, feat, loop_weight, bias2d)


def kernel(feat, edge_index, etypes, weight, loop_weight, bias):
    npad = E_PAD - N_EDGES
    src = jnp.concatenate(
        [edge_index[0].astype(jnp.int32), jnp.zeros((npad,), jnp.int32)])
    dst = jnp.concatenate(
        [edge_index[1].astype(jnp.int32),
         jnp.full((npad,), N_PAD - 1, jnp.int32)])
    et = jnp.concatenate(
        [etypes.astype(jnp.int32), jnp.zeros((npad,), jnp.int32)])
    zeros = jnp.zeros((N_PAD, D), jnp.float32)

    xw = _make_xw(feat, weight)
    xw_flat = xw.reshape(NUM_RELS * N_NODES, D)
    part = _run_sc(xw_flat, src, et, dst, zeros)
    bias2d = bias.reshape(1, D)
    return _combine(part, feat, loop_weight, bias2d)
